# Initial kernel scaffold; baseline (speedup 1.0000x reference)
import jax
import jax.numpy as jnp
from jax.experimental import pallas as pl


def kernel(x, pos_table):
    raise NotImplementedError("write your pallas kernel here")


# speedup vs baseline: 1.4472x; 1.4472x over previous
import jax
import jax.numpy as jnp
from jax.experimental import pallas as pl

_MBLK = 512


def _add_body(x_ref, p_ref, o_ref):
    o_ref[...] = x_ref[...] + p_ref[...]


def kernel(x, pos_table):
    B, M, D = x.shape
    x = jnp.reshape(x, (B, M, D))
    grid = (M // _MBLK, B)
    return pl.pallas_call(
        _add_body,
        grid=grid,
        in_specs=[
            pl.BlockSpec((1, _MBLK, D), lambda i, b: (b, i, 0)),
            pl.BlockSpec((_MBLK, D), lambda i, b: (i, 0)),
        ],
        out_specs=pl.BlockSpec((1, _MBLK, D), lambda i, b: (b, i, 0)),
        out_shape=jax.ShapeDtypeStruct((B, M, D), x.dtype),
    )(x, pos_table)
